# baseline (speedup 1.0000x reference)
import functools

import jax
import jax.numpy as jnp
from jax import lax
from jax.experimental import pallas as pl
from jax.experimental.pallas import tpu as pltpu
from jax.experimental.pallas import tpu_sc as plsc

N_NODES = 10000
N_EDGES = 320000
D = 128
N_CLS = 40

NC = 2
NS = 16
NW = NC * NS
CHUNK = 128
NCHUNK = 80
EPW = NCHUNK * CHUNK
E_PAD = NW * EPW
N_ACC = N_NODES + 256
RPT = 624
TAIL = N_NODES - NS * RPT

_mesh = plsc.VectorSubcoreMesh(core_axis_name="c", subcore_axis_name="s")


def _sc_agg_body(with_deg, *refs):
    if with_deg:
        (x_hbm, pk_hbm, z2_hbm, z1_hbm,
         agg_hbm, deg_hbm,
         pk_v, sc0_v, sc1_v, dc_v, rows_v0, rows_v1, ones_v,
         acc_sh, deg_sh, sem0, sem1) = refs
    else:
        (x_hbm, pk_hbm, z2_hbm,
         agg_hbm,
         pk_v, sc0_v, sc1_v, dc_v, rows_v0, rows_v1,
         acc_sh, sem0, sem1) = refs

    cid = lax.axis_index("c")
    sid = lax.axis_index("s")
    wid = cid * NS + sid

    pltpu.sync_copy(z2_hbm.at[pl.ds(sid * RPT, RPT)],
                    acc_sh.at[pl.ds(sid * RPT, RPT)])

    @pl.when(sid == NS - 1)
    def _():
        pltpu.sync_copy(z2_hbm.at[pl.ds(NS * RPT, N_ACC - NS * RPT)],
                        acc_sh.at[pl.ds(NS * RPT, N_ACC - NS * RPT)])
    if with_deg:
        @pl.when(sid == 0)
        def _():
            pltpu.sync_copy(z1_hbm, deg_sh)
        for i in range(CHUNK // 16):
            ones_v[pl.ds(i * 16, 16)] = jnp.full((16,), 1.0, jnp.float32)

    pltpu.sync_copy(pk_hbm.at[wid], pk_v)

    def unpack_src(j, buf):
        for i in range(CHUNK // 16):
            v = pk_v[j, pl.ds(i * 16, 16)]
            buf[pl.ds(i * 16, 16)] = lax.bitwise_and(v, 0xFFFF)

    def unpack_dst(j, buf):
        for i in range(CHUNK // 16):
            v = pk_v[j, pl.ds(i * 16, 16)]
            buf[pl.ds(i * 16, 16)] = lax.shift_right_logical(v, 16)

    unpack_src(0, sc0_v)
    unpack_src(1, sc1_v)
    pltpu.async_copy(x_hbm.at[sc0_v], rows_v0, sem0)
    pltpu.async_copy(x_hbm.at[sc1_v], rows_v1, sem1)

    plsc.subcore_barrier()

    def drain(buf, sem):
        pltpu.make_async_copy(x_hbm.at[pl.ds(0, CHUNK)], buf, sem).wait()

    def consume(j, buf):
        unpack_dst(j, dc_v)
        pltpu.sync_copy(buf, acc_sh.at[dc_v], add=True)
        if with_deg:
            pltpu.sync_copy(ones_v, deg_sh.at[dc_v], add=True)

    def step(k, carry):
        j0 = 2 * k
        drain(rows_v0, sem0)
        consume(j0, rows_v0)
        unpack_src(j0 + 2, sc0_v)
        pltpu.async_copy(x_hbm.at[sc0_v], rows_v0, sem0)
        j1 = 2 * k + 1
        drain(rows_v1, sem1)
        consume(j1, rows_v1)
        unpack_src(j1 + 2, sc1_v)
        pltpu.async_copy(x_hbm.at[sc1_v], rows_v1, sem1)
        return carry

    lax.fori_loop(0, NCHUNK // 2 - 1, step, 0)

    drain(rows_v0, sem0)
    consume(NCHUNK - 2, rows_v0)
    drain(rows_v1, sem1)
    consume(NCHUNK - 1, rows_v1)

    plsc.subcore_barrier()

    pltpu.sync_copy(acc_sh.at[pl.ds(sid * RPT, RPT)],
                    agg_hbm.at[cid, pl.ds(sid * RPT, RPT)])

    @pl.when(sid == NS - 1)
    def _():
        pltpu.sync_copy(acc_sh.at[pl.ds(NS * RPT, TAIL)],
                        agg_hbm.at[cid, pl.ds(NS * RPT, TAIL)])
    if with_deg:
        @pl.when(sid == 0)
        def _():
            pltpu.sync_copy(deg_sh, deg_hbm.at[cid])


_sc_agg_deg = functools.partial(
    pl.kernel,
    functools.partial(_sc_agg_body, True),
    mesh=_mesh,
    out_type=[
        jax.ShapeDtypeStruct((NC, N_NODES, D), jnp.float32),
        jax.ShapeDtypeStruct((NC, N_ACC), jnp.float32),
    ],
    scratch_types=[
        pltpu.VMEM((NCHUNK, CHUNK), jnp.int32),
        pltpu.VMEM((CHUNK,), jnp.int32),
        pltpu.VMEM((CHUNK,), jnp.int32),
        pltpu.VMEM((CHUNK,), jnp.int32),
        pltpu.VMEM((CHUNK, D), jnp.float32),
        pltpu.VMEM((CHUNK, D), jnp.float32),
        pltpu.VMEM((CHUNK,), jnp.float32),
        pltpu.VMEM_SHARED((N_ACC, D), jnp.float32),
        pltpu.VMEM_SHARED((N_ACC,), jnp.float32),
        pltpu.SemaphoreType.DMA,
        pltpu.SemaphoreType.DMA,
    ],
)()

_sc_agg = functools.partial(
    pl.kernel,
    functools.partial(_sc_agg_body, False),
    mesh=_mesh,
    out_type=jax.ShapeDtypeStruct((NC, N_NODES, D), jnp.float32),
    scratch_types=[
        pltpu.VMEM((NCHUNK, CHUNK), jnp.int32),
        pltpu.VMEM((CHUNK,), jnp.int32),
        pltpu.VMEM((CHUNK,), jnp.int32),
        pltpu.VMEM((CHUNK,), jnp.int32),
        pltpu.VMEM((CHUNK, D), jnp.float32),
        pltpu.VMEM((CHUNK, D), jnp.float32),
        pltpu.VMEM_SHARED((N_ACC, D), jnp.float32),
        pltpu.SemaphoreType.DMA,
        pltpu.SemaphoreType.DMA,
    ],
)()


BLK = 1000


def _row_spec():
    return pl.BlockSpec((BLK, D), lambda i: (i, 0))


def _tc_self_body(h_ref, ws_ref, o_ref):
    o_ref[...] = jnp.dot(h_ref[...], ws_ref[...],
                         preferred_element_type=jnp.float32)


_tc_self = pl.pallas_call(
    _tc_self_body,
    grid=(N_NODES // BLK,),
    in_specs=[_row_spec(), pl.BlockSpec((D, D), lambda i: (0, 0))],
    out_specs=_row_spec(),
    out_shape=jax.ShapeDtypeStruct((N_NODES, D), jnp.float32),
)


def _tc_comb_body(final, *refs):
    if final:
        (xs_ref, a0_ref, a1_ref, d0_ref, d1_ref,
         wn_ref, b_ref, wfc_ref, bfc_ref, o_ref) = refs
    else:
        (xs_ref, a0_ref, a1_ref, d0_ref, d1_ref,
         wn_ref, b_ref, o_ref) = refs
    deg = d0_ref[...] + d1_ref[...]
    inv = 1.0 / jnp.maximum(deg, 1.0)
    hn = (a0_ref[...] + a1_ref[...]) * inv
    h = (xs_ref[...]
         + jnp.dot(hn, wn_ref[...], preferred_element_type=jnp.float32)
         + b_ref[...])
    h = jnp.maximum(h, 0.0)
    if final:
        h = (jnp.dot(h, wfc_ref[...], preferred_element_type=jnp.float32)
             + bfc_ref[...])
    o_ref[...] = h


def _tc_comb(final):
    in_specs = [
        _row_spec(),
        _row_spec(),
        _row_spec(),
        pl.BlockSpec((BLK, 1), lambda i: (i, 0)),
        pl.BlockSpec((BLK, 1), lambda i: (i, 0)),
        pl.BlockSpec((D, D), lambda i: (0, 0)),
        pl.BlockSpec((1, D), lambda i: (0, 0)),
    ]
    if final:
        in_specs += [
            pl.BlockSpec((D, D), lambda i: (0, 0)),
            pl.BlockSpec((1, D), lambda i: (0, 0)),
        ]
    return pl.pallas_call(
        functools.partial(_tc_comb_body, final),
        grid=(N_NODES // BLK,),
        in_specs=in_specs,
        out_specs=_row_spec(),
        out_shape=jax.ShapeDtypeStruct((N_NODES, D), jnp.float32),
    )


_tc_mid = _tc_comb(False)
_tc_fin = _tc_comb(True)


def kernel(x, edge_index, W_self1, W_neigh1, b1, W_self2, W_neigh2, b2,
           W_fc, b_fc):
    pad_w = EPW - N_EDGES // NW
    real = ((edge_index[1].astype(jnp.int32) << 16)
            | edge_index[0].astype(jnp.int32)).reshape(NW, N_EDGES // NW)
    pad_i = jnp.arange(pad_w, dtype=jnp.int32)[None, :]
    pad_w_id = jnp.arange(NW, dtype=jnp.int32)[:, None]
    pad_src = (pad_w_id * 311 + pad_i * 97) % N_NODES
    pads = ((N_NODES + (pad_w_id % NS) * NS + (pad_i % NS)) << 16) | pad_src
    packed = jnp.concatenate([real, pads], axis=1).reshape(NW, NCHUNK, CHUNK)
    z2 = jnp.zeros((N_ACC, D), jnp.float32)
    z1 = jnp.zeros((N_ACC,), jnp.float32)

    agg1, degp = _sc_agg_deg(x, packed, z2, z1)
    xs1 = _tc_self(x, W_self1)
    d0 = degp[0, :N_NODES].reshape(N_NODES, 1)
    d1 = degp[1, :N_NODES].reshape(N_NODES, 1)
    b1r = b1.reshape(1, D)
    b2r = b2.reshape(1, D)

    h1 = _tc_mid(xs1, agg1[0], agg1[1], d0, d1, W_neigh1, b1r)

    agg2 = _sc_agg(h1, packed, z2)
    xs2 = _tc_self(h1, W_self2)

    wfc_pad = jnp.zeros((D, D), jnp.float32).at[:, :N_CLS].set(W_fc)
    bfc_pad = jnp.zeros((1, D), jnp.float32).at[0, :N_CLS].set(b_fc)
    out_pad = _tc_fin(xs2, agg2[0], agg2[1], d0, d1, W_neigh2, b2r,
                      wfc_pad, bfc_pad)
    return out_pad[:, :N_CLS]


# speedup vs baseline: 1.0374x; 1.0374x over previous
import functools

import jax
import jax.numpy as jnp
from jax import lax
from jax.experimental import pallas as pl
from jax.experimental.pallas import tpu as pltpu
from jax.experimental.pallas import tpu_sc as plsc

N_NODES = 10000
N_EDGES = 320000
D = 128
N_CLS = 40

NC = 2
NS = 16
NW = NC * NS
CHUNK = 128
NCHUNK = 80
EPW = NCHUNK * CHUNK
E_PAD = NW * EPW
N_ACC = N_NODES + 256
RPT = 624
TAIL = N_NODES - NS * RPT

_mesh = plsc.VectorSubcoreMesh(core_axis_name="c", subcore_axis_name="s")


def _sc_agg_body(with_deg, *refs):
    if with_deg:
        (x_hbm, pk_hbm, z2_hbm, z1_hbm,
         agg0_hbm, agg1_hbm, deg0_hbm, deg1_hbm,
         pk_v, sc0_v, sc1_v, dc_v, rows_v0, rows_v1, ones_v,
         acc_sh, deg_sh, sem0, sem1) = refs
    else:
        (x_hbm, pk_hbm, z2_hbm,
         agg0_hbm, agg1_hbm,
         pk_v, sc0_v, sc1_v, dc_v, rows_v0, rows_v1,
         acc_sh, sem0, sem1) = refs

    cid = lax.axis_index("c")
    sid = lax.axis_index("s")
    wid = cid * NS + sid

    pltpu.sync_copy(z2_hbm.at[pl.ds(sid * RPT, RPT)],
                    acc_sh.at[pl.ds(sid * RPT, RPT)])

    @pl.when(sid == NS - 1)
    def _():
        pltpu.sync_copy(z2_hbm.at[pl.ds(NS * RPT, N_ACC - NS * RPT)],
                        acc_sh.at[pl.ds(NS * RPT, N_ACC - NS * RPT)])
    if with_deg:
        @pl.when(sid == 0)
        def _():
            pltpu.sync_copy(z1_hbm, deg_sh)
        for i in range(CHUNK // 16):
            ones_v[pl.ds(i * 16, 16)] = jnp.full((16,), 1.0, jnp.float32)

    pltpu.sync_copy(pk_hbm.at[wid], pk_v)

    def unpack_src(j, buf):
        for i in range(CHUNK // 16):
            v = pk_v[j, pl.ds(i * 16, 16)]
            buf[pl.ds(i * 16, 16)] = lax.bitwise_and(v, 0xFFFF)

    def unpack_dst(j, buf):
        for i in range(CHUNK // 16):
            v = pk_v[j, pl.ds(i * 16, 16)]
            buf[pl.ds(i * 16, 16)] = lax.shift_right_logical(v, 16)

    unpack_src(0, sc0_v)
    unpack_src(1, sc1_v)
    pltpu.async_copy(x_hbm.at[sc0_v], rows_v0, sem0)
    pltpu.async_copy(x_hbm.at[sc1_v], rows_v1, sem1)

    plsc.subcore_barrier()

    def drain(buf, sem):
        pltpu.make_async_copy(x_hbm.at[pl.ds(0, CHUNK)], buf, sem).wait()

    def consume(j, buf):
        unpack_dst(j, dc_v)
        pltpu.sync_copy(buf, acc_sh.at[dc_v], add=True)
        if with_deg:
            pltpu.sync_copy(ones_v, deg_sh.at[dc_v], add=True)

    def step(k, carry):
        j0 = 2 * k
        drain(rows_v0, sem0)
        consume(j0, rows_v0)
        unpack_src(j0 + 2, sc0_v)
        pltpu.async_copy(x_hbm.at[sc0_v], rows_v0, sem0)
        j1 = 2 * k + 1
        drain(rows_v1, sem1)
        consume(j1, rows_v1)
        unpack_src(j1 + 2, sc1_v)
        pltpu.async_copy(x_hbm.at[sc1_v], rows_v1, sem1)
        return carry

    lax.fori_loop(0, NCHUNK // 2 - 1, step, 0)

    drain(rows_v0, sem0)
    consume(NCHUNK - 2, rows_v0)
    drain(rows_v1, sem1)
    consume(NCHUNK - 1, rows_v1)

    plsc.subcore_barrier()

    for c, agg_hbm in ((0, agg0_hbm), (1, agg1_hbm)):
        @pl.when(cid == c)
        def _():
            pltpu.sync_copy(acc_sh.at[pl.ds(sid * RPT, RPT)],
                            agg_hbm.at[pl.ds(sid * RPT, RPT)])

            @pl.when(sid == NS - 1)
            def _():
                pltpu.sync_copy(acc_sh.at[pl.ds(NS * RPT, TAIL)],
                                agg_hbm.at[pl.ds(NS * RPT, TAIL)])
    if with_deg:
        for c, deg_hbm in ((0, deg0_hbm), (1, deg1_hbm)):
            @pl.when((cid == c) & (sid == 0))
            def _():
                pltpu.sync_copy(deg_sh, deg_hbm)


_sc_agg_deg = functools.partial(
    pl.kernel,
    functools.partial(_sc_agg_body, True),
    mesh=_mesh,
    out_type=[
        jax.ShapeDtypeStruct((N_NODES, D), jnp.float32),
        jax.ShapeDtypeStruct((N_NODES, D), jnp.float32),
        jax.ShapeDtypeStruct((N_ACC,), jnp.float32),
        jax.ShapeDtypeStruct((N_ACC,), jnp.float32),
    ],
    scratch_types=[
        pltpu.VMEM((NCHUNK, CHUNK), jnp.int32),
        pltpu.VMEM((CHUNK,), jnp.int32),
        pltpu.VMEM((CHUNK,), jnp.int32),
        pltpu.VMEM((CHUNK,), jnp.int32),
        pltpu.VMEM((CHUNK, D), jnp.float32),
        pltpu.VMEM((CHUNK, D), jnp.float32),
        pltpu.VMEM((CHUNK,), jnp.float32),
        pltpu.VMEM_SHARED((N_ACC, D), jnp.float32),
        pltpu.VMEM_SHARED((N_ACC,), jnp.float32),
        pltpu.SemaphoreType.DMA,
        pltpu.SemaphoreType.DMA,
    ],
)()

_sc_agg = functools.partial(
    pl.kernel,
    functools.partial(_sc_agg_body, False),
    mesh=_mesh,
    out_type=[
        jax.ShapeDtypeStruct((N_NODES, D), jnp.float32),
        jax.ShapeDtypeStruct((N_NODES, D), jnp.float32),
    ],
    scratch_types=[
        pltpu.VMEM((NCHUNK, CHUNK), jnp.int32),
        pltpu.VMEM((CHUNK,), jnp.int32),
        pltpu.VMEM((CHUNK,), jnp.int32),
        pltpu.VMEM((CHUNK,), jnp.int32),
        pltpu.VMEM((CHUNK, D), jnp.float32),
        pltpu.VMEM((CHUNK, D), jnp.float32),
        pltpu.VMEM_SHARED((N_ACC, D), jnp.float32),
        pltpu.SemaphoreType.DMA,
        pltpu.SemaphoreType.DMA,
    ],
)()


BLK = 1000


def _row_spec():
    return pl.BlockSpec((BLK, D), lambda i: (i, 0))


def _tc_self_body(h_ref, ws_ref, o_ref):
    o_ref[...] = jnp.dot(h_ref[...], ws_ref[...],
                         preferred_element_type=jnp.float32)


_tc_self = pl.pallas_call(
    _tc_self_body,
    grid=(N_NODES // BLK,),
    in_specs=[_row_spec(), pl.BlockSpec((D, D), lambda i: (0, 0))],
    out_specs=_row_spec(),
    out_shape=jax.ShapeDtypeStruct((N_NODES, D), jnp.float32),
)


def _tc_comb_body(final, *refs):
    if final:
        (xs_ref, a0_ref, a1_ref, d0_ref, d1_ref,
         wn_ref, b_ref, wfc_ref, bfc_ref, o_ref) = refs
    else:
        (xs_ref, a0_ref, a1_ref, d0_ref, d1_ref,
         wn_ref, b_ref, o_ref) = refs
    deg = d0_ref[...] + d1_ref[...]
    inv = 1.0 / jnp.maximum(deg, 1.0)
    hn = (a0_ref[...] + a1_ref[...]) * inv
    h = (xs_ref[...]
         + jnp.dot(hn, wn_ref[...], preferred_element_type=jnp.float32)
         + b_ref[...])
    h = jnp.maximum(h, 0.0)
    if final:
        h = (jnp.dot(h, wfc_ref[...], preferred_element_type=jnp.float32)
             + bfc_ref[...])
    o_ref[...] = h


def _tc_comb(final):
    in_specs = [
        _row_spec(),
        _row_spec(),
        _row_spec(),
        pl.BlockSpec((BLK, 1), lambda i: (i, 0)),
        pl.BlockSpec((BLK, 1), lambda i: (i, 0)),
        pl.BlockSpec((D, D), lambda i: (0, 0)),
        pl.BlockSpec((1, D), lambda i: (0, 0)),
    ]
    if final:
        in_specs += [
            pl.BlockSpec((D, N_CLS), lambda i: (0, 0)),
            pl.BlockSpec((1, N_CLS), lambda i: (0, 0)),
        ]
    out_d = N_CLS if final else D
    return pl.pallas_call(
        functools.partial(_tc_comb_body, final),
        grid=(N_NODES // BLK,),
        in_specs=in_specs,
        out_specs=pl.BlockSpec((BLK, out_d), lambda i: (i, 0)),
        out_shape=jax.ShapeDtypeStruct((N_NODES, out_d), jnp.float32),
    )


_tc_mid = _tc_comb(False)
_tc_fin = _tc_comb(True)


def kernel(x, edge_index, W_self1, W_neigh1, b1, W_self2, W_neigh2, b2,
           W_fc, b_fc):
    pad_w = EPW - N_EDGES // NW
    real = ((edge_index[1].astype(jnp.int32) << 16)
            | edge_index[0].astype(jnp.int32)).reshape(NW, N_EDGES // NW)
    pad_i = jnp.arange(pad_w, dtype=jnp.int32)[None, :]
    pad_w_id = jnp.arange(NW, dtype=jnp.int32)[:, None]
    pad_src = (pad_w_id * 311 + pad_i * 97) % N_NODES
    pads = ((N_NODES + (pad_w_id % NS) * NS + (pad_i % NS)) << 16) | pad_src
    packed = jnp.concatenate([real, pads], axis=1).reshape(NW, NCHUNK, CHUNK)
    z2 = jnp.zeros((N_ACC, D), jnp.float32)
    z1 = jnp.zeros((N_ACC,), jnp.float32)

    a0, a1, deg0, deg1 = _sc_agg_deg(x, packed, z2, z1)
    xs1 = _tc_self(x, W_self1)
    d0 = deg0.reshape(N_ACC, 1)
    d1 = deg1.reshape(N_ACC, 1)
    b1r = b1.reshape(1, D)
    b2r = b2.reshape(1, D)

    h1 = _tc_mid(xs1, a0, a1, d0, d1, W_neigh1, b1r)

    b0, b1a = _sc_agg(h1, packed, z2)
    xs2 = _tc_self(h1, W_self2)

    return _tc_fin(xs2, b0, b1a, d0, d1, W_neigh2, b2r,
                   W_fc, b_fc.reshape(1, N_CLS))
